# Initial kernel scaffold; baseline (speedup 1.0000x reference)
import jax
import jax.numpy as jnp
from jax.experimental import pallas as pl


def kernel(x1, edge_index1, edge_attr1, node2graph1, x2, edge_index2, edge_attr2, node2graph2, mc, params):
    raise NotImplementedError("write your pallas kernel here")


# speedup vs baseline: 3.9633x; 3.9633x over previous
import functools

import jax
import jax.numpy as jnp
from jax import lax
from jax.experimental import pallas as pl
from jax.experimental.pallas import tpu as pltpu
from jax.experimental.pallas import tpu_sc as plsc

N = 8192
E = 16384
G = 256
D = 64
ED = 12
EH = 12
HID = 256
STEPS = 3
TS = 6

NW = 32
BPW = 2 * E // NW
CH = BPW // 128
NBLK = N // 512

@functools.cache
def _sc_kernels():
    mesh = plsc.VectorSubcoreMesh(core_axis_name="c", subcore_axis_name="s")

    @functools.partial(
        pl.kernel,
        out_type=jax.ShapeDtypeStruct((2 * E, D), jnp.float32),
        mesh=mesh,
        scratch_types=[
            pltpu.VMEM((CH, 128), jnp.int32),
            pltpu.VMEM((BPW, D), jnp.float32),
            pltpu.SemaphoreType.DMA,
        ],
        compiler_params=pltpu.CompilerParams(use_tc_tiling_on_sc=False),
    )
    def sc_gather(table_hbm, idx_hbm, out_hbm, idx_v, rows_v, sem):
        wid = lax.axis_index("c") * 16 + lax.axis_index("s")
        base = wid * BPW
        pltpu.sync_copy(idx_hbm.at[wid], idx_v)
        cps = []
        for j in range(CH):
            cps.append(pltpu.async_copy(
                table_hbm.at[idx_v.at[j]], rows_v.at[pl.ds(j * 128, 128)],
                sem))
        for c in cps:
            c.wait()
        pltpu.sync_copy(rows_v, out_hbm.at[pl.ds(base, BPW)])

    @functools.partial(
        pl.kernel,
        out_type=jax.ShapeDtypeStruct((2 * N, D), jnp.float32),
        mesh=mesh,
        scratch_types=[
            pltpu.VMEM((CH, 128), jnp.int32),
            pltpu.VMEM((BPW, D), jnp.float32),
            pltpu.VMEM_SHARED((N, D), jnp.float32),
        ],
        compiler_params=pltpu.CompilerParams(use_tc_tiling_on_sc=False),
    )
    def sc_scatter(msg_hbm, idx_hbm, zero_hbm, out_hbm, idx_v, msg_v, acc_sh):
        c = lax.axis_index("c")
        s = lax.axis_index("s")
        wid = c * 16 + s
        pltpu.sync_copy(zero_hbm, acc_sh.at[pl.ds(s * 512, 512)])
        plsc.subcore_barrier()
        pltpu.sync_copy(msg_hbm.at[pl.ds(wid * BPW, BPW)], msg_v)
        pltpu.sync_copy(idx_hbm.at[wid], idx_v)
        for j in range(CH):
            pltpu.sync_copy(msg_v.at[pl.ds(j * 128, 128)],
                            acc_sh.at[idx_v.at[j]], add=True)
        plsc.subcore_barrier()
        pltpu.sync_copy(acc_sh.at[pl.ds(s * 512, 512)],
                        out_hbm.at[pl.ds(c * N + s * 512, 512)])

    return sc_gather, sc_scatter


def _dot(a, b):
    return lax.dot_general(a, b, (((1,), (0,)), ((), ())),
                           preferred_element_type=jnp.float32)


def _proj_body(x_ref, w_ref, b_ref, o_ref):
    o_ref[...] = jax.nn.relu(_dot(x_ref[...], w_ref[0]) + b_ref[0, 0])


def _proj(xs, w_s, b_s):
    return pl.pallas_call(
        _proj_body,
        grid=(2, NBLK),
        in_specs=[
            pl.BlockSpec((512, D), lambda b, i: (b * NBLK + i, 0)),
            pl.BlockSpec((1, D, D), lambda b, i: (b, 0, 0)),
            pl.BlockSpec((1, 1, D), lambda b, i: (b, 0, 0)),
        ],
        out_specs=pl.BlockSpec((512, D), lambda b, i: (b * NBLK + i, 0)),
        out_shape=jax.ShapeDtypeStruct((2 * N, D), jnp.float32),
    )(xs, w_s, b_s)


def _msg_body(hs_ref, ea_ref, w1_ref, b1_ref, t_ref, bm_ref, o_ref):
    hs = hs_ref[...]
    zed = jax.nn.relu(_dot(ea_ref[...], w1_ref[0]) + b1_ref[0, 0])
    msg = _dot(hs, bm_ref[0])
    for h in range(EH):
        sel = lax.broadcasted_iota(jnp.int32, (1, EH), 1) == h
        zh = jnp.sum(zed * jnp.where(sel, 1.0, 0.0), axis=1, keepdims=True)
        msg = msg + zh * _dot(hs, t_ref[0, h])
    o_ref[...] = msg


def _msg(hs, ea_s, w1_s, b1_s, t_s, bm_s):
    eblk = 2 * E // 512
    return pl.pallas_call(
        _msg_body,
        grid=(2, eblk // 2),
        in_specs=[
            pl.BlockSpec((512, D), lambda b, i: (b * (eblk // 2) + i, 0)),
            pl.BlockSpec((512, ED), lambda b, i: (b * (eblk // 2) + i, 0)),
            pl.BlockSpec((1, ED, EH), lambda b, i: (b, 0, 0)),
            pl.BlockSpec((1, 1, EH), lambda b, i: (b, 0, 0)),
            pl.BlockSpec((1, EH, D, D), lambda b, i: (b, 0, 0, 0)),
            pl.BlockSpec((1, D, D), lambda b, i: (b, 0, 0)),
        ],
        out_specs=pl.BlockSpec((512, D), lambda b, i: (b * (eblk // 2) + i, 0)),
        out_shape=jax.ShapeDtypeStruct((2 * E, D), jnp.float32),
    )(hs, ea_s, w1_s, b1_s, t_s, bm_s)


def _gru_math(x, h, wir, wiz, win, whr, whz, whn, bir, biz, bin_, bhr, bhz, bhn):
    r = jax.nn.sigmoid(_dot(x, wir) + bir + _dot(h, whr) + bhr)
    z = jax.nn.sigmoid(_dot(x, wiz) + biz + _dot(h, whz) + bhz)
    n = jnp.tanh(_dot(x, win) + bin_ + r * (_dot(h, whn) + bhn))
    return (1.0 - z) * n + z * h


def _gru_body(a_ref, h_ref, bc_ref, wi_ref, wh_ref, bi_ref, bh_ref, o_ref):
    m = jax.nn.relu(a_ref[...] + bc_ref[0, 0])
    h = h_ref[...]
    o_ref[...] = _gru_math(
        m, h, wi_ref[0, 0], wi_ref[0, 1], wi_ref[0, 2],
        wh_ref[0, 0], wh_ref[0, 1], wh_ref[0, 2],
        bi_ref[0, 0, 0], bi_ref[0, 1, 0], bi_ref[0, 2, 0],
        bh_ref[0, 0, 0], bh_ref[0, 1, 0], bh_ref[0, 2, 0])


def _gru_step(agg, hid, bc_s, wi_s, wh_s, bi_s, bh_s):
    return pl.pallas_call(
        _gru_body,
        grid=(2, NBLK),
        in_specs=[
            pl.BlockSpec((512, D), lambda b, i: (b * NBLK + i, 0)),
            pl.BlockSpec((512, D), lambda b, i: (b * NBLK + i, 0)),
            pl.BlockSpec((1, 1, D), lambda b, i: (b, 0, 0)),
            pl.BlockSpec((1, 3, D, D), lambda b, i: (b, 0, 0, 0)),
            pl.BlockSpec((1, 3, D, D), lambda b, i: (b, 0, 0, 0)),
            pl.BlockSpec((1, 3, 1, D), lambda b, i: (b, 0, 0, 0)),
            pl.BlockSpec((1, 3, 1, D), lambda b, i: (b, 0, 0, 0)),
        ],
        out_specs=pl.BlockSpec((512, D), lambda b, i: (b * NBLK + i, 0)),
        out_shape=jax.ShapeDtypeStruct((2 * N, D), jnp.float32),
    )(agg, hid, bc_s, wi_s, wh_s, bi_s, bh_s)


def _readout_body(nf_ref, ngc_ref, ngr_ref, wl_ref, bl_ref, wn_ref, bn_ref,
                  wi_ref, wh_ref, bi_ref, bh_ref, g_ref,
                  lg_sc, ms_sc, ss_sc, gf_sc, gr_sc):
    def oh_block(i):
        col = ngc_ref[0, pl.ds(i * 512, 512), :]
        io = lax.broadcasted_iota(jnp.int32, (512, G), 1).astype(jnp.float32)
        return jnp.where(io == col, 1.0, 0.0)

    def oht_block(i):
        row = ngr_ref[0, i, :].reshape(1, 512)
        io = lax.broadcasted_iota(jnp.int32, (G, 512), 0).astype(jnp.float32)
        return jnp.where(io == row, 1.0, 0.0)

    gf_sc[...] = jnp.zeros((G, D), jnp.float32)
    def init_blk(i, _):
        nf = nf_ref[pl.ds(i * 512, 512), :]
        gf_sc[...] += _dot(oht_block(i), nf)
        return 0
    lax.fori_loop(0, NBLK, init_blk, 0)

    for ts in range(TS):
        gf = gf_sc[...]
        gv = _dot(jax.nn.relu(gf), wl_ref[0, ts, 0:D, :])
        wln = wl_ref[0, ts, D:2 * D, :]
        blv = bl_ref[0, 0, ts]

        ms_sc[...] = jnp.full((1, G), -1e30, jnp.float32)
        def p1(i, _):
            oh = oh_block(i)
            nf = nf_ref[pl.ds(i * 512, 512), :]
            lg = _dot(oh, gv) + _dot(nf, wln) + blv
            lg = jnp.where(lg >= 0, lg, 0.01 * lg)
            lg_sc[pl.ds(i * 512, 512), :] = lg
            masked = jnp.where(oh > 0.5, lg, -1e30)
            ms_sc[...] = jnp.maximum(ms_sc[...],
                                     jnp.max(masked, axis=0, keepdims=True))
            return 0
        lax.fori_loop(0, NBLK, p1, 0)
        msv = ms_sc[...]
        msv = jnp.where(msv < -1e29, 0.0, msv)

        ss_sc[...] = jnp.zeros((1, G), jnp.float32)
        def p2(i, _):
            oh = oh_block(i)
            lg = lg_sc[pl.ds(i * 512, 512), :]
            mb = jnp.sum(oh * msv, axis=1, keepdims=True)
            ex = jnp.exp(lg - mb)
            lg_sc[pl.ds(i * 512, 512), :] = ex
            ss_sc[...] += jnp.sum(oh * ex, axis=0, keepdims=True)
            return 0
        lax.fori_loop(0, NBLK, p2, 0)
        ssv = ss_sc[...]

        gr_sc[...] = jnp.zeros((G, D), jnp.float32)
        wn = wn_ref[0, ts]
        bn = bn_ref[0, ts]
        def p3(i, _):
            oh = oh_block(i)
            nf = nf_ref[pl.ds(i * 512, 512), :]
            ex = lg_sc[pl.ds(i * 512, 512), :]
            sb = jnp.sum(oh * ssv, axis=1, keepdims=True)
            a = ex / sb
            hv = _dot(nf, wn) + bn
            gr_sc[...] += _dot(oht_block(i), a * hv)
            return 0
        lax.fori_loop(0, NBLK, p3, 0)
        grep = gr_sc[...]
        grep = jnp.where(grep > 0, grep, jnp.exp(jnp.minimum(grep, 0.0)) - 1.0)

        gf_sc[...] = _gru_math(
            grep, gf, wi_ref[0, ts, 0], wi_ref[0, ts, 1], wi_ref[0, ts, 2],
            wh_ref[0, ts, 0], wh_ref[0, ts, 1], wh_ref[0, ts, 2],
            bi_ref[0, ts, 0, 0], bi_ref[0, ts, 1, 0], bi_ref[0, ts, 2, 0],
            bh_ref[0, ts, 0, 0], bh_ref[0, ts, 1, 0], bh_ref[0, ts, 2, 0])

    g_ref[0] = gf_sc[...]


def _readout(nf, ngc, ngr, wl_s, bl_s, wn_s, bn_s, wi_s, wh_s, bi_s, bh_s):
    return pl.pallas_call(
        _readout_body,
        grid=(2,),
        in_specs=[
            pl.BlockSpec((N, D), lambda b: (b, 0)),
            pl.BlockSpec((1, N, 1), lambda b: (b, 0, 0)),
            pl.BlockSpec((1, NBLK, 512), lambda b: (b, 0, 0)),
            pl.BlockSpec((1, TS, 2 * D, 1), lambda b: (b, 0, 0, 0)),
            pl.BlockSpec((1, 1, TS), lambda b: (b, 0, 0)),
            pl.BlockSpec((1, TS, D, D), lambda b: (b, 0, 0, 0)),
            pl.BlockSpec((1, TS, 1, D), lambda b: (b, 0, 0, 0)),
            pl.BlockSpec((1, TS, 3, D, D), lambda b: (b, 0, 0, 0, 0)),
            pl.BlockSpec((1, TS, 3, D, D), lambda b: (b, 0, 0, 0, 0)),
            pl.BlockSpec((1, TS, 3, 1, D), lambda b: (b, 0, 0, 0, 0)),
            pl.BlockSpec((1, TS, 3, 1, D), lambda b: (b, 0, 0, 0, 0)),
        ],
        out_specs=pl.BlockSpec((1, G, D), lambda b: (b, 0, 0)),
        out_shape=jax.ShapeDtypeStruct((2, G, D), jnp.float32),
        scratch_shapes=[
            pltpu.VMEM((N, 1), jnp.float32),
            pltpu.VMEM((1, G), jnp.float32),
            pltpu.VMEM((1, G), jnp.float32),
            pltpu.VMEM((G, D), jnp.float32),
            pltpu.VMEM((G, D), jnp.float32),
        ],
    )(nf, ngc, ngr, wl_s, bl_s, wn_s, bn_s, wi_s, wh_s, bi_s, bh_s)


def _pred_body(g_ref, mc_ref, w1a_ref, w1b_ref, w1c_ref, b1_ref,
               gm_ref, bt_ref, w2_ref, b2_ref, o_ref):
    h = jax.nn.relu(_dot(g_ref[0], w1a_ref[...]) + _dot(g_ref[1], w1b_ref[...])
                    + _dot(mc_ref[...], w1c_ref[...]) + b1_ref[0])
    h = h * (gm_ref[0] / jnp.sqrt(1.0 + 1e-5)) + bt_ref[0]
    o_ref[...] = _dot(h, w2_ref[...]) + b2_ref[0]


def _pred(g_out, mc, w1a, w1b, w1c, b1, gm, bt, w2, b2):
    return pl.pallas_call(
        _pred_body,
        out_shape=jax.ShapeDtypeStruct((G, 2), jnp.float32),
    )(g_out, mc, w1a, w1b, w1c, b1, gm, bt, w2, b2)


def _stack(p1, p2, name):
    return jnp.stack([p1[name], p2[name]])


def kernel(x1, edge_index1, edge_attr1, node2graph1,
           x2, edge_index2, edge_attr2, node2graph2, mc, params):
    m1, m2 = params['mpnn1'], params['mpnn2']

    xs = jnp.concatenate([x1, x2], axis=0)
    ea_s = jnp.concatenate([edge_attr1, edge_attr2], axis=0)
    src_g = jnp.concatenate([edge_index1[0], edge_index2[0] + N])
    src3 = src_g.reshape(NW, CH, 128)
    dst3 = jnp.concatenate([edge_index1[1], edge_index2[1]]).reshape(NW, CH, 128)
    zeros512 = jnp.zeros((512, D), jnp.float32)

    wp_s = _stack(m1, m2, 'Wp')
    bp_s = _stack(m1, m2, 'bp').reshape(2, 1, D)
    w1_s = _stack(m1, m2, 'W1')
    b1_s = _stack(m1, m2, 'b1').reshape(2, 1, EH)
    t_s = jnp.stack([m1['W2'].reshape(EH, D, D), m2['W2'].reshape(EH, D, D)])
    bm_s = jnp.stack([m1['b2'].reshape(D, D), m2['b2'].reshape(D, D)])
    bc_s = _stack(m1, m2, 'b_conv').reshape(2, 1, D)

    def gru_mats(g):
        wi = g['W_ih'].reshape(3, D, D)
        wh = g['W_hh'].reshape(3, D, D)
        return (jnp.transpose(wi, (0, 2, 1)), jnp.transpose(wh, (0, 2, 1)),
                g['b_ih'].reshape(3, 1, D), g['b_hh'].reshape(3, 1, D))

    g1 = gru_mats(m1['gru'])
    g2 = gru_mats(m2['gru'])
    wi_s, wh_s, bi_s, bh_s = (jnp.stack([a, b]) for a, b in zip(g1, g2))

    def ro_stack(plist, f):
        return jnp.stack([f(p) for p in plist])

    def ro_params(plist):
        wl = ro_stack(plist, lambda p: p['Wl'])
        bl = ro_stack(plist, lambda p: p['bl']).reshape(1, TS)
        wn = ro_stack(plist, lambda p: p['Wn'])
        bn = ro_stack(plist, lambda p: p['bn']).reshape(TS, 1, D)
        gw = [gru_mats(p['gru']) for p in plist]
        rwi = jnp.stack([g[0] for g in gw])
        rwh = jnp.stack([g[1] for g in gw])
        rbi = jnp.stack([g[2] for g in gw])
        rbh = jnp.stack([g[3] for g in gw])
        return wl, bl, wn, bn, rwi, rwh, rbi, rbh

    ro1 = ro_params(params['ro1'])
    ro2 = ro_params(params['ro2'])
    (rwl, rbl, rwn, rbn, rwi, rwh, rbi, rbh) = (
        jnp.stack([a, b]) for a, b in zip(ro1, ro2))

    ng_s = jnp.stack([node2graph1, node2graph2]).astype(jnp.float32)
    ngc = ng_s.reshape(2, N, 1)
    ngr = ng_s.reshape(2, NBLK, 512)

    sc_gather, sc_scatter = _sc_kernels()
    node = _proj(xs, wp_s, bp_s)
    hidden = node
    for _ in range(STEPS):
        hs = sc_gather(node, src3)
        msg = _msg(hs, ea_s, w1_s, b1_s, t_s, bm_s)
        agg = sc_scatter(msg, dst3, zeros512)
        node = _gru_step(agg, hidden, bc_s, wi_s, wh_s, bi_s, bh_s)
        hidden = node

    g_out = _readout(node, ngc, ngr, rwl, rbl, rwn, rbn, rwi, rwh, rbi, rbh)

    pp = params['pred']
    w1a = pp['W1'][0:D]
    w1b = pp['W1'][D:2 * D]
    w1c = pp['W1'][2 * D:]
    return _pred(g_out, mc, w1a, w1b, w1c, pp['b1'].reshape(1, HID),
                 pp['gamma'].reshape(1, HID), pp['beta'].reshape(1, HID),
                 pp['W2'], pp['b2'].reshape(1, 2))


# speedup vs baseline: 4.6104x; 1.1633x over previous
import functools

import jax
import jax.numpy as jnp
from jax import lax
from jax.experimental import pallas as pl
from jax.experimental.pallas import tpu as pltpu
from jax.experimental.pallas import tpu_sc as plsc

N = 8192
E = 16384
G = 256
D = 64
ED = 12
EH = 12
HID = 256
STEPS = 3
TS = 6

NW = 32
BPW = 2 * E // NW
CH = BPW // 128
NBLK = N // 512
RBLK = 1024
NRB = N // RBLK


@functools.cache
def _sc_kernels():
    mesh = plsc.VectorSubcoreMesh(core_axis_name="c", subcore_axis_name="s")

    @functools.partial(
        pl.kernel,
        out_type=jax.ShapeDtypeStruct((2 * E, D), jnp.float32),
        mesh=mesh,
        scratch_types=[
            pltpu.VMEM((CH, 128), jnp.int32),
            pltpu.VMEM((BPW, D), jnp.float32),
            pltpu.SemaphoreType.DMA,
        ],
        compiler_params=pltpu.CompilerParams(use_tc_tiling_on_sc=False),
    )
    def sc_gather(table_hbm, idx_hbm, out_hbm, idx_v, rows_v, sem):
        wid = lax.axis_index("c") * 16 + lax.axis_index("s")
        base = wid * BPW
        pltpu.sync_copy(idx_hbm.at[wid], idx_v)
        cps = []
        for j in range(CH):
            cps.append(pltpu.async_copy(
                table_hbm.at[idx_v.at[j]], rows_v.at[pl.ds(j * 128, 128)],
                sem))
        for c in cps:
            c.wait()
        pltpu.sync_copy(rows_v, out_hbm.at[pl.ds(base, BPW)])

    @functools.partial(
        pl.kernel,
        out_type=jax.ShapeDtypeStruct((2 * N, D), jnp.float32),
        mesh=mesh,
        scratch_types=[
            pltpu.VMEM((CH, 128), jnp.int32),
            pltpu.VMEM((BPW, D), jnp.float32),
            pltpu.VMEM_SHARED((N, D), jnp.float32),
        ],
        compiler_params=pltpu.CompilerParams(use_tc_tiling_on_sc=False),
    )
    def sc_scatter(msg_hbm, idx_hbm, zero_hbm, out_hbm, idx_v, msg_v, acc_sh):
        c = lax.axis_index("c")
        s = lax.axis_index("s")
        wid = c * 16 + s
        pltpu.sync_copy(zero_hbm, acc_sh.at[pl.ds(s * 512, 512)])
        plsc.subcore_barrier()
        pltpu.sync_copy(msg_hbm.at[pl.ds(wid * BPW, BPW)], msg_v)
        pltpu.sync_copy(idx_hbm.at[wid], idx_v)
        for j in range(CH):
            pltpu.sync_copy(msg_v.at[pl.ds(j * 128, 128)],
                            acc_sh.at[idx_v.at[j]], add=True)
        plsc.subcore_barrier()
        pltpu.sync_copy(acc_sh.at[pl.ds(s * 512, 512)],
                        out_hbm.at[pl.ds(c * N + s * 512, 512)])

    return sc_gather, sc_scatter


def _dot(a, b):
    return lax.dot_general(a, b, (((1,), (0,)), ((), ())),
                           preferred_element_type=jnp.float32)


def _dot_t(a, b):
    return lax.dot_general(a, b, (((1,), (1,)), ((), ())),
                           preferred_element_type=jnp.float32)


def _dot_lt(a, b):
    return lax.dot_general(a, b, (((0,), (0,)), ((), ())),
                           preferred_element_type=jnp.float32)


def _proj_body(x_ref, w_ref, b_ref, o_ref):
    o_ref[...] = jax.nn.relu(_dot(x_ref[...], w_ref[0]) + b_ref[0, 0])


def _proj(xs, w_s, b_s):
    return pl.pallas_call(
        _proj_body,
        grid=(2, NBLK),
        in_specs=[
            pl.BlockSpec((512, D), lambda b, i: (b * NBLK + i, 0)),
            pl.BlockSpec((1, D, D), lambda b, i: (b, 0, 0)),
            pl.BlockSpec((1, 1, D), lambda b, i: (b, 0, 0)),
        ],
        out_specs=pl.BlockSpec((512, D), lambda b, i: (b * NBLK + i, 0)),
        out_shape=jax.ShapeDtypeStruct((2 * N, D), jnp.float32),
    )(xs, w_s, b_s)


def _msg_body(hs_ref, ea_ref, w1_ref, b1_ref, t2_ref, bm_ref, r_ref, s_ref,
              o_ref):
    hs = hs_ref[...]
    zed = jax.nn.relu(_dot(ea_ref[...], w1_ref[0]) + b1_ref[0, 0])
    p = _dot(hs, t2_ref[0])
    z = _dot(zed, r_ref[...])
    o_ref[...] = _dot(z * p, s_ref[...]) + _dot(hs, bm_ref[0])


def _msg(hs, ea_s, w1_s, b1_s, t2_s, bm_s, rmat, smat):
    eb = 2 * E // 512
    return pl.pallas_call(
        _msg_body,
        grid=(2, eb // 2),
        in_specs=[
            pl.BlockSpec((512, D), lambda b, i: (b * (eb // 2) + i, 0)),
            pl.BlockSpec((512, ED), lambda b, i: (b * (eb // 2) + i, 0)),
            pl.BlockSpec((1, ED, EH), lambda b, i: (b, 0, 0)),
            pl.BlockSpec((1, 1, EH), lambda b, i: (b, 0, 0)),
            pl.BlockSpec((1, D, EH * D), lambda b, i: (b, 0, 0)),
            pl.BlockSpec((1, D, D), lambda b, i: (b, 0, 0)),
            pl.BlockSpec((EH, EH * D), lambda b, i: (0, 0)),
            pl.BlockSpec((EH * D, D), lambda b, i: (0, 0)),
        ],
        out_specs=pl.BlockSpec((512, D), lambda b, i: (b * (eb // 2) + i, 0)),
        out_shape=jax.ShapeDtypeStruct((2 * E, D), jnp.float32),
    )(hs, ea_s, w1_s, b1_s, t2_s, bm_s, rmat, smat)


def _gru_math(x, h, wi, wh, bi, bh):
    r = jax.nn.sigmoid(_dot_t(x, wi[0]) + bi[0] + _dot_t(h, wh[0]) + bh[0])
    z = jax.nn.sigmoid(_dot_t(x, wi[1]) + bi[1] + _dot_t(h, wh[1]) + bh[1])
    n = jnp.tanh(_dot_t(x, wi[2]) + bi[2] + r * (_dot_t(h, wh[2]) + bh[2]))
    return (1.0 - z) * n + z * h


def _gru_body(a_ref, h_ref, bc_ref, wi_ref, wh_ref, bi_ref, bh_ref, o_ref):
    m = jax.nn.relu(a_ref[...] + bc_ref[0, 0])
    o_ref[...] = _gru_math(m, h_ref[...], wi_ref[0], wh_ref[0],
                           bi_ref[0], bh_ref[0])


def _gru_step(agg, hid, bc_s, wi_s, wh_s, bi_s, bh_s):
    return pl.pallas_call(
        _gru_body,
        grid=(2, NBLK),
        in_specs=[
            pl.BlockSpec((512, D), lambda b, i: (b * NBLK + i, 0)),
            pl.BlockSpec((512, D), lambda b, i: (b * NBLK + i, 0)),
            pl.BlockSpec((1, 1, D), lambda b, i: (b, 0, 0)),
            pl.BlockSpec((1, 3, D, D), lambda b, i: (b, 0, 0, 0)),
            pl.BlockSpec((1, 3, D, D), lambda b, i: (b, 0, 0, 0)),
            pl.BlockSpec((1, 3, 1, D), lambda b, i: (b, 0, 0, 0)),
            pl.BlockSpec((1, 3, 1, D), lambda b, i: (b, 0, 0, 0)),
        ],
        out_specs=pl.BlockSpec((512, D), lambda b, i: (b * NBLK + i, 0)),
        out_shape=jax.ShapeDtypeStruct((2 * N, D), jnp.float32),
    )(agg, hid, bc_s, wi_s, wh_s, bi_s, bh_s)


def _readout_body(nf_ref, ngc_ref, *refs):
    wrefs = refs[:8 * TS]
    g_ref = refs[8 * TS]
    oh_sc, lg_sc, ms_sc, ss_sc, gf_sc, gr_sc = refs[8 * TS + 1:]

    gf_sc[...] = jnp.zeros((G, D), jnp.float32)
    def init_blk(i, _):
        sl = pl.ds(i * RBLK, RBLK)
        col = ngc_ref[sl, :]
        io = lax.broadcasted_iota(jnp.int32, (RBLK, G), 1).astype(jnp.float32)
        oh = jnp.where(io == col, 1.0, 0.0)
        oh_sc[sl, :] = oh
        gf_sc[...] += _dot_lt(oh, nf_ref[sl, :])
        return 0
    lax.fori_loop(0, NRB, init_blk, 0)

    for ts in range(TS):
        (wl_ref, bl_ref, wn_ref, bn_ref,
         wi_ref, wh_ref, bi_ref, bh_ref) = wrefs[8 * ts:8 * ts + 8]
        gf = gf_sc[...]
        gv = _dot(jax.nn.relu(gf), wl_ref[0:D, :])
        wln = wl_ref[D:2 * D, :]
        blv = bl_ref[0, 0]

        ms_sc[...] = jnp.full((1, G), -1e30, jnp.float32)
        def p1(i, _):
            sl = pl.ds(i * RBLK, RBLK)
            oh = oh_sc[sl, :]
            lg = _dot(oh, gv) + _dot(nf_ref[sl, :], wln) + blv
            lg = jnp.where(lg >= 0, lg, 0.01 * lg)
            lg_sc[sl, :] = lg
            masked = jnp.where(oh > 0.5, lg, -1e30)
            ms_sc[...] = jnp.maximum(ms_sc[...],
                                     jnp.max(masked, axis=0, keepdims=True))
            return 0
        lax.fori_loop(0, NRB, p1, 0)
        msv = ms_sc[...]
        msv = jnp.where(msv < -1e29, 0.0, msv)

        ss_sc[...] = jnp.zeros((1, G), jnp.float32)
        def p2(i, _):
            sl = pl.ds(i * RBLK, RBLK)
            oh = oh_sc[sl, :]
            lg = lg_sc[sl, :]
            mb = jnp.sum(oh * msv, axis=1, keepdims=True)
            ex = jnp.exp(lg - mb)
            lg_sc[sl, :] = ex
            ss_sc[...] += jnp.sum(oh * ex, axis=0, keepdims=True)
            return 0
        lax.fori_loop(0, NRB, p2, 0)
        ssv = ss_sc[...]

        gr_sc[...] = jnp.zeros((G, D), jnp.float32)
        def p3(i, _):
            sl = pl.ds(i * RBLK, RBLK)
            oh = oh_sc[sl, :]
            ex = lg_sc[sl, :]
            sb = jnp.sum(oh * ssv, axis=1, keepdims=True)
            a = ex / sb
            hv = _dot(nf_ref[sl, :], wn_ref[...]) + bn_ref[...]
            gr_sc[...] += _dot_lt(oh, a * hv)
            return 0
        lax.fori_loop(0, NRB, p3, 0)
        grep = gr_sc[...]
        grep = jnp.where(grep > 0, grep, jnp.exp(jnp.minimum(grep, 0.0)) - 1.0)

        gf_sc[...] = _gru_math(grep, gf, wi_ref, wh_ref, bi_ref, bh_ref)

    g_ref[...] = gf_sc[...]


def _readout(nf_b, ngc_b, wts):
    return pl.pallas_call(
        _readout_body,
        out_shape=jax.ShapeDtypeStruct((G, D), jnp.float32),
        scratch_shapes=[
            pltpu.VMEM((N, G), jnp.float32),
            pltpu.VMEM((N, 1), jnp.float32),
            pltpu.VMEM((1, G), jnp.float32),
            pltpu.VMEM((1, G), jnp.float32),
            pltpu.VMEM((G, D), jnp.float32),
            pltpu.VMEM((G, D), jnp.float32),
        ],
    )(nf_b, ngc_b, *wts)


def _pred_body(g1_ref, g2_ref, mc_ref, w1a_ref, w1b_ref, w1c_ref, b1_ref,
               gm_ref, bt_ref, w2_ref, b2_ref, o_ref):
    h = jax.nn.relu(_dot(g1_ref[...], w1a_ref[...])
                    + _dot(g2_ref[...], w1b_ref[...])
                    + _dot(mc_ref[...], w1c_ref[...]) + b1_ref[...])
    h = h * (gm_ref[...] / jnp.sqrt(1.0 + 1e-5)) + bt_ref[...]
    o_ref[...] = _dot(h, w2_ref[...]) + b2_ref[...]


def _pred(g1, g2, mc, w1a, w1b, w1c, b1, gm, bt, w2, b2):
    return pl.pallas_call(
        _pred_body,
        out_shape=jax.ShapeDtypeStruct((G, 2), jnp.float32),
    )(g1, g2, mc, w1a, w1b, w1c, b1, gm, bt, w2, b2)


def kernel(x1, edge_index1, edge_attr1, node2graph1,
           x2, edge_index2, edge_attr2, node2graph2, mc, params):
    m1, m2 = params['mpnn1'], params['mpnn2']

    xs = jnp.concatenate([x1, x2], axis=0)
    ea_s = jnp.concatenate([edge_attr1, edge_attr2], axis=0)
    src_g = jnp.concatenate([edge_index1[0], edge_index2[0] + N])
    src3 = src_g.reshape(NW, CH, 128)
    dst3 = jnp.concatenate([edge_index1[1], edge_index2[1]]).reshape(NW, CH, 128)
    zeros512 = jnp.zeros((512, D), jnp.float32)

    wp_s = jnp.stack([m1['Wp'], m2['Wp']])
    bp_s = jnp.stack([m1['bp'], m2['bp']]).reshape(2, 1, D)
    w1_s = jnp.stack([m1['W1'], m2['W1']])
    b1_s = jnp.stack([m1['b1'], m2['b1']]).reshape(2, 1, EH)
    def t2(m):
        return m['W2'].reshape(EH, D, D).transpose(1, 0, 2).reshape(D, EH * D)

    t2_s = jnp.stack([t2(m1), t2(m2)])
    bm_s = jnp.stack([m1['b2'].reshape(D, D), m2['b2'].reshape(D, D)])
    rmat = jnp.repeat(jnp.eye(EH, dtype=jnp.float32), D, axis=1)
    smat = jnp.tile(jnp.eye(D, dtype=jnp.float32), (EH, 1))
    bc_s = jnp.stack([m1['b_conv'], m2['b_conv']]).reshape(2, 1, D)
    wi_s = jnp.stack([m1['gru']['W_ih'], m2['gru']['W_ih']]).reshape(2, 3, D, D)
    wh_s = jnp.stack([m1['gru']['W_hh'], m2['gru']['W_hh']]).reshape(2, 3, D, D)
    bi_s = jnp.stack([m1['gru']['b_ih'], m2['gru']['b_ih']]).reshape(2, 3, 1, D)
    bh_s = jnp.stack([m1['gru']['b_hh'], m2['gru']['b_hh']]).reshape(2, 3, 1, D)

    def ro_wts(plist):
        wts = []
        for p in plist:
            wts += [p['Wl'], p['bl'].reshape(1, 1), p['Wn'],
                    p['bn'].reshape(1, D),
                    p['gru']['W_ih'].reshape(3, D, D),
                    p['gru']['W_hh'].reshape(3, D, D),
                    p['gru']['b_ih'].reshape(3, 1, D),
                    p['gru']['b_hh'].reshape(3, 1, D)]
        return wts

    sc_gather, sc_scatter = _sc_kernels()
    node = _proj(xs, wp_s, bp_s)
    hidden = node
    for _ in range(STEPS):
        hs = sc_gather(node, src3)
        msg = _msg(hs, ea_s, w1_s, b1_s, t2_s, bm_s, rmat, smat)
        agg = sc_scatter(msg, dst3, zeros512)
        node = _gru_step(agg, hidden, bc_s, wi_s, wh_s, bi_s, bh_s)
        hidden = node

    nf1 = lax.slice(node, (0, 0), (N, D))
    nf2 = lax.slice(node, (N, 0), (2 * N, D))
    r1 = _readout(nf1, node2graph1.astype(jnp.float32).reshape(N, 1),
                  ro_wts(params['ro1']))
    r2 = _readout(nf2, node2graph2.astype(jnp.float32).reshape(N, 1),
                  ro_wts(params['ro2']))

    pp = params['pred']
    return _pred(r1, r2, mc,
                 pp['W1'][0:D], pp['W1'][D:2 * D], pp['W1'][2 * D:],
                 pp['b1'].reshape(1, HID), pp['gamma'].reshape(1, HID),
                 pp['beta'].reshape(1, HID), pp['W2'], pp['b2'].reshape(1, 2))


# speedup vs baseline: 6.4967x; 1.4091x over previous
import functools

import jax
import jax.numpy as jnp
from jax import lax
from jax.experimental import pallas as pl
from jax.experimental.pallas import tpu as pltpu
from jax.experimental.pallas import tpu_sc as plsc

N = 8192
E = 16384
G = 256
D = 64
ED = 12
EH = 12
HID = 256
STEPS = 3
TS = 6

W = 128
NW = 32
BPW = 2 * E // NW
CH = BPW // 128
RBLK = 1024
NBLK = N // RBLK
NRB = NBLK


@functools.cache
def _sc_kernels():
    mesh = plsc.VectorSubcoreMesh(core_axis_name="c", subcore_axis_name="s")

    @functools.partial(
        pl.kernel,
        out_type=jax.ShapeDtypeStruct((2 * E, W), jnp.float32),
        mesh=mesh,
        scratch_types=[
            pltpu.VMEM((CH, 128), jnp.int32),
            pltpu.VMEM((512, W), jnp.float32),
            pltpu.SemaphoreType.DMA,
        ],
        compiler_params=pltpu.CompilerParams(use_tc_tiling_on_sc=True),
    )
    def sc_gather(table_hbm, idx_hbm, out_hbm, idx_v, rows_v, sem):
        wid = lax.axis_index("c") * 16 + lax.axis_index("s")
        pltpu.sync_copy(idx_hbm.at[wid], idx_v)
        for half in range(2):
            cps = []
            for j in range(CH // 2):
                cps.append(pltpu.async_copy(
                    table_hbm.at[idx_v.at[half * (CH // 2) + j]],
                    rows_v.at[pl.ds(j * 128, 128)], sem))
            for c in cps:
                c.wait()
            pltpu.sync_copy(
                rows_v, out_hbm.at[pl.ds(wid * BPW + half * 512, 512)])

    @functools.partial(
        pl.kernel,
        out_type=jax.ShapeDtypeStruct((2 * N, D), jnp.float32),
        mesh=mesh,
        scratch_types=[
            pltpu.VMEM((CH, 128), jnp.int32),
            pltpu.VMEM((512, D), jnp.float32),
            pltpu.VMEM_SHARED((N, D), jnp.float32),
        ],
        compiler_params=pltpu.CompilerParams(use_tc_tiling_on_sc=False),
    )
    def sc_scatter(msg_hbm, idx_hbm, zero_hbm, out_hbm, idx_v, msg_v, acc_sh):
        c = lax.axis_index("c")
        s = lax.axis_index("s")
        wid = c * 16 + s
        for q in range(4):
            pltpu.sync_copy(zero_hbm,
                            acc_sh.at[pl.ds(s * 512 + q * 128, 128)])
        plsc.subcore_barrier()
        pltpu.sync_copy(idx_hbm.at[wid], idx_v)
        for half in range(2):
            pltpu.sync_copy(
                msg_hbm.at[pl.ds(wid * BPW + half * 512, 512), pl.ds(0, D)],
                msg_v)
            for j in range(CH // 2):
                pltpu.sync_copy(msg_v.at[pl.ds(j * 128, 128)],
                                acc_sh.at[idx_v.at[half * (CH // 2) + j]],
                                add=True)
        plsc.subcore_barrier()
        pltpu.sync_copy(acc_sh.at[pl.ds(s * 512, 512)],
                        out_hbm.at[pl.ds(c * N + s * 512, 512)])

    return sc_gather, sc_scatter


def _dot(a, b):
    return lax.dot_general(a, b, (((1,), (0,)), ((), ())),
                           preferred_element_type=jnp.float32)


def _dot_t(a, b):
    return lax.dot_general(a, b, (((1,), (1,)), ((), ())),
                           preferred_element_type=jnp.float32)


def _proj_body(x_ref, w_ref, b_ref, o_ref):
    o_ref[...] = jax.nn.relu(_dot(x_ref[...], w_ref[0]) + b_ref[0, 0])


def _proj(xs, w_s, b_s):
    nb = 2 * N // RBLK
    return pl.pallas_call(
        _proj_body,
        grid=(2, nb // 2),
        in_specs=[
            pl.BlockSpec((RBLK, D), lambda b, i: (b * (nb // 2) + i, 0)),
            pl.BlockSpec((1, D, W), lambda b, i: (b, 0, 0)),
            pl.BlockSpec((1, 1, W), lambda b, i: (b, 0, 0)),
        ],
        out_specs=pl.BlockSpec((RBLK, W), lambda b, i: (b * (nb // 2) + i, 0)),
        out_shape=jax.ShapeDtypeStruct((2 * N, W), jnp.float32),
    )(xs, w_s, b_s)


def _msg_body(hs_ref, ea_ref, w1_ref, b1_ref, t2_ref, bm_ref, r_ref, s_ref,
              o_ref):
    hs = hs_ref[:, 0:D]
    zed = jax.nn.relu(_dot(ea_ref[...], w1_ref[0]) + b1_ref[0, 0])
    p = _dot(hs, t2_ref[0])
    z = _dot(zed, r_ref[...])
    o_ref[...] = _dot(z * p, s_ref[...]) + _dot(hs, bm_ref[0])


def _msg(hs, ea_s, w1_s, b1_s, t2_s, bm_s, rmat, smat):
    eb = 2 * E // RBLK
    return pl.pallas_call(
        _msg_body,
        grid=(2, eb // 2),
        in_specs=[
            pl.BlockSpec((RBLK, W), lambda b, i: (b * (eb // 2) + i, 0)),
            pl.BlockSpec((RBLK, ED), lambda b, i: (b * (eb // 2) + i, 0)),
            pl.BlockSpec((1, ED, EH), lambda b, i: (b, 0, 0)),
            pl.BlockSpec((1, 1, EH), lambda b, i: (b, 0, 0)),
            pl.BlockSpec((1, D, EH * D), lambda b, i: (b, 0, 0)),
            pl.BlockSpec((1, D, W), lambda b, i: (b, 0, 0)),
            pl.BlockSpec((EH, EH * D), lambda b, i: (0, 0)),
            pl.BlockSpec((EH * D, W), lambda b, i: (0, 0)),
        ],
        out_specs=pl.BlockSpec((RBLK, W), lambda b, i: (b * (eb // 2) + i, 0)),
        out_shape=jax.ShapeDtypeStruct((2 * E, W), jnp.float32),
    )(hs, ea_s, w1_s, b1_s, t2_s, bm_s, rmat, smat)


def _gru_math(x, h, wi, wh, bi, bh):
    r = jax.nn.sigmoid(_dot_t(x, wi[0:D]) + bi[:, 0:D]
                       + _dot_t(h, wh[0:D]) + bh[:, 0:D])
    z = jax.nn.sigmoid(_dot_t(x, wi[D:2 * D]) + bi[:, D:2 * D]
                       + _dot_t(h, wh[D:2 * D]) + bh[:, D:2 * D])
    n = jnp.tanh(_dot_t(x, wi[2 * D:]) + bi[:, 2 * D:]
                 + r * (_dot_t(h, wh[2 * D:]) + bh[:, 2 * D:]))
    return (1.0 - z) * n + z * h


def _gru_body(a_ref, h_ref, bc_ref, wi_ref, wh_ref, bi_ref, bh_ref, o_ref):
    m = jax.nn.relu(a_ref[...] + bc_ref[0, 0])
    h = h_ref[:, 0:D]
    out = _gru_math(m, h, wi_ref[0], wh_ref[0], bi_ref[0], bh_ref[0])
    o_ref[:, 0:D] = out
    o_ref[:, D:W] = jnp.zeros((RBLK, W - D), jnp.float32)


def _gru_step(agg, hid, bc_s, wi_s, wh_s, bi_s, bh_s):
    nb = 2 * N // RBLK
    return pl.pallas_call(
        _gru_body,
        grid=(2, nb // 2),
        in_specs=[
            pl.BlockSpec((RBLK, D), lambda b, i: (b * (nb // 2) + i, 0)),
            pl.BlockSpec((RBLK, W), lambda b, i: (b * (nb // 2) + i, 0)),
            pl.BlockSpec((1, 1, D), lambda b, i: (b, 0, 0)),
            pl.BlockSpec((1, 3 * D, D), lambda b, i: (b, 0, 0)),
            pl.BlockSpec((1, 3 * D, D), lambda b, i: (b, 0, 0)),
            pl.BlockSpec((1, 1, 3 * D), lambda b, i: (b, 0, 0)),
            pl.BlockSpec((1, 1, 3 * D), lambda b, i: (b, 0, 0)),
        ],
        out_specs=pl.BlockSpec((RBLK, W), lambda b, i: (b * (nb // 2) + i, 0)),
        out_shape=jax.ShapeDtypeStruct((2 * N, W), jnp.float32),
    )(agg, hid, bc_s, wi_s, wh_s, bi_s, bh_s)


def _readout_body(nf_ref, ng_ref, *refs):
    wrefs = refs[:8 * TS]
    g_ref = refs[8 * TS]
    oht_sc, lg_sc, ms_sc, ss_sc, gf_sc, gr_sc = refs[8 * TS + 1:]

    gf_sc[...] = jnp.zeros((G, D), jnp.float32)
    def init_blk(i, _):
        row = ng_ref[pl.ds(i, 1), :]
        io = lax.broadcasted_iota(jnp.int32, (G, RBLK), 0).astype(jnp.float32)
        oht = jnp.where(io == row, 1.0, 0.0)
        oht_sc[pl.ds(i, 1)] = oht.reshape(1, G, RBLK)
        gf_sc[...] += _dot(oht, nf_ref[pl.ds(i * RBLK, RBLK), 0:D])
        return 0
    lax.fori_loop(0, NRB, init_blk, 0)

    for ts in range(TS):
        (wl_ref, bl_ref, wn_ref, bn_ref,
         wi_ref, wh_ref, bi_ref, bh_ref) = wrefs[8 * ts:8 * ts + 8]
        gf = gf_sc[...]
        dcol = lambda c, m_: lax.dot_general(
            c, m_, (((0,), (1,)), ((), ())),
            preferred_element_type=jnp.float32)
        gv = dcol(wl_ref[0:D, :], jax.nn.relu(gf))
        blv = bl_ref[0, 0]

        ms_sc[...] = jnp.full((G, 1), -1e30, jnp.float32)
        def p1(i, _):
            oht = oht_sc[pl.ds(i, 1)].reshape(G, RBLK)
            nf = nf_ref[pl.ds(i * RBLK, RBLK), 0:D]
            lg = _dot(gv, oht) + dcol(wl_ref[D:2 * D, :], nf) + blv

            lg = jnp.where(lg >= 0, lg, 0.01 * lg)
            lg_sc[pl.ds(i, 1), :] = lg
            masked = jnp.where(oht > 0.5, lg, -1e30)
            ms_sc[...] = jnp.maximum(ms_sc[...],
                                     jnp.max(masked, axis=1, keepdims=True))
            return 0
        lax.fori_loop(0, NRB, p1, 0)
        msv = ms_sc[...]
        msv = jnp.where(msv < -1e29, 0.0, msv)

        ss_sc[...] = jnp.zeros((G, 1), jnp.float32)
        def p2(i, _):
            oht = oht_sc[pl.ds(i, 1)].reshape(G, RBLK)
            lg = lg_sc[pl.ds(i, 1), :]
            mb = jnp.sum(oht * msv, axis=0, keepdims=True)
            ex = jnp.exp(lg - mb)
            lg_sc[pl.ds(i, 1), :] = ex
            ss_sc[...] += jnp.sum(oht * ex, axis=1, keepdims=True)
            return 0
        lax.fori_loop(0, NRB, p2, 0)
        ssv = ss_sc[...]

        gr_sc[...] = jnp.zeros((G, D), jnp.float32)
        def p3(i, _):
            oht = oht_sc[pl.ds(i, 1)].reshape(G, RBLK)
            ex = lg_sc[pl.ds(i, 1), :]
            sb = jnp.sum(oht * ssv, axis=0, keepdims=True)
            a = ex / sb
            nf = nf_ref[pl.ds(i * RBLK, RBLK), 0:D]
            hv = _dot(nf, wn_ref[...]) + bn_ref[...]
            gr_sc[...] += _dot(oht * a, hv)
            return 0
        lax.fori_loop(0, NRB, p3, 0)
        grep = gr_sc[...]
        grep = jnp.where(grep > 0, grep, jnp.exp(jnp.minimum(grep, 0.0)) - 1.0)

        gf_sc[...] = _gru_math(grep, gf, wi_ref, wh_ref,
                               bi_ref[...], bh_ref[...])

    g_ref[...] = gf_sc[...]


def _readout(node, branch, ng8, wts):
    nspec = [
        pl.BlockSpec((N, W), lambda g: (branch, 0)),
        pl.BlockSpec((NRB, RBLK), lambda g: (0, 0)),
    ] + [pl.BlockSpec(w.shape, lambda g, nd=w.ndim: (0,) * nd) for w in wts]
    return pl.pallas_call(
        _readout_body,
        grid=(1,),
        in_specs=nspec,
        out_specs=pl.BlockSpec((G, D), lambda g: (0, 0)),
        out_shape=jax.ShapeDtypeStruct((G, D), jnp.float32),
        scratch_shapes=[
            pltpu.VMEM((NRB, G, RBLK), jnp.float32),
            pltpu.VMEM((NRB, RBLK), jnp.float32),
            pltpu.VMEM((G, 1), jnp.float32),
            pltpu.VMEM((G, 1), jnp.float32),
            pltpu.VMEM((G, D), jnp.float32),
            pltpu.VMEM((G, D), jnp.float32),
        ],
    )(node, ng8, *wts)


def _pred_body(g1_ref, g2_ref, mc_ref, w1a_ref, w1b_ref, w1c_ref, b1_ref,
               gm_ref, bt_ref, w2_ref, b2_ref, o_ref):
    h = jax.nn.relu(_dot(g1_ref[...], w1a_ref[...])
                    + _dot(g2_ref[...], w1b_ref[...])
                    + _dot(mc_ref[...], w1c_ref[...]) + b1_ref[...])
    h = h * (gm_ref[...] / jnp.sqrt(1.0 + 1e-5)) + bt_ref[...]
    o_ref[...] = _dot(h, w2_ref[...]) + b2_ref[...]


def _pred(g1, g2, mc, w1a, w1b, w1c, b1, gm, bt, w2, b2):
    return pl.pallas_call(
        _pred_body,
        out_shape=jax.ShapeDtypeStruct((G, 2), jnp.float32),
    )(g1, g2, mc, w1a, w1b, w1c, b1, gm, bt, w2, b2)


def kernel(x1, edge_index1, edge_attr1, node2graph1,
           x2, edge_index2, edge_attr2, node2graph2, mc, params):
    m1, m2 = params['mpnn1'], params['mpnn2']
    f32 = jnp.float32

    xs = jnp.concatenate([x1, x2], axis=0)
    ea_s = jnp.concatenate([edge_attr1, edge_attr2], axis=0)
    src3 = jnp.concatenate(
        [edge_index1[0], edge_index2[0] + N]).reshape(NW, CH, 128)
    dst3 = jnp.concatenate(
        [edge_index1[1], edge_index2[1]]).reshape(NW, CH, 128)
    zeros128 = jnp.zeros((128, D), f32)

    def padw(a):
        return jnp.pad(a, ((0, 0), (0, W - D)))

    wp_s = jnp.stack([padw(m1['Wp']), padw(m2['Wp'])])
    bp_s = jnp.stack([jnp.pad(m1['bp'], (0, W - D)),
                      jnp.pad(m2['bp'], (0, W - D))]).reshape(2, 1, W)
    w1_s = jnp.stack([m1['W1'], m2['W1']])
    b1_s = jnp.stack([m1['b1'], m2['b1']]).reshape(2, 1, EH)

    def t2(m):
        return m['W2'].reshape(EH, D, D).transpose(1, 0, 2).reshape(D, EH * D)

    t2_s = jnp.stack([t2(m1), t2(m2)])
    bm_s = jnp.stack([padw(m1['b2'].reshape(D, D)),
                      padw(m2['b2'].reshape(D, D))])
    rmat = jnp.repeat(jnp.eye(EH, dtype=f32), D, axis=1)
    smat = jnp.tile(jnp.pad(jnp.eye(D, dtype=f32), ((0, 0), (0, W - D))),
                    (EH, 1))
    bc_s = jnp.stack([m1['b_conv'], m2['b_conv']]).reshape(2, 1, D)
    wi_s = jnp.stack([m1['gru']['W_ih'], m2['gru']['W_ih']])
    wh_s = jnp.stack([m1['gru']['W_hh'], m2['gru']['W_hh']])
    bi_s = jnp.stack([m1['gru']['b_ih'], m2['gru']['b_ih']]).reshape(2, 1, 3 * D)
    bh_s = jnp.stack([m1['gru']['b_hh'], m2['gru']['b_hh']]).reshape(2, 1, 3 * D)

    def ro_wts(plist):
        wts = []
        for p in plist:
            wts += [p['Wl'], p['bl'].reshape(1, 1), p['Wn'],
                    p['bn'].reshape(1, D),
                    p['gru']['W_ih'], p['gru']['W_hh'],
                    p['gru']['b_ih'].reshape(1, 3 * D),
                    p['gru']['b_hh'].reshape(1, 3 * D)]
        return wts

    sc_gather, sc_scatter = _sc_kernels()
    node = _proj(xs, wp_s, bp_s)
    hidden = node
    for _ in range(STEPS):
        hs = sc_gather(node, src3)
        msg = _msg(hs, ea_s, w1_s, b1_s, t2_s, bm_s, rmat, smat)
        agg = sc_scatter(msg, dst3, zeros128)
        node = _gru_step(agg, hidden, bc_s, wi_s, wh_s, bi_s, bh_s)
        hidden = node

    ng81 = node2graph1.astype(f32).reshape(NRB, RBLK)
    ng82 = node2graph2.astype(f32).reshape(NRB, RBLK)
    r1 = _readout(node, 0, ng81, ro_wts(params['ro1']))
    r2 = _readout(node, 1, ng82, ro_wts(params['ro2']))

    pp = params['pred']
    return _pred(r1, r2, mc,
                 pp['W1'][0:D], pp['W1'][D:2 * D], pp['W1'][2 * D:],
                 pp['b1'].reshape(1, HID), pp['gamma'].reshape(1, HID),
                 pp['beta'].reshape(1, HID), pp['W2'], pp['b2'].reshape(1, 2))


# speedup vs baseline: 6.9008x; 1.0622x over previous
import functools

import jax
import jax.numpy as jnp
from jax import lax
from jax.experimental import pallas as pl
from jax.experimental.pallas import tpu as pltpu
from jax.experimental.pallas import tpu_sc as plsc

N = 8192
E = 16384
G = 256
D = 64
ED = 12
EH = 12
HID = 256
STEPS = 3
TS = 6

W = 128
NW = 32
BPW = 2 * E // NW
CH = BPW // 128
RBLK = 1024
NBLK = N // RBLK
NRB = NBLK


@functools.cache
def _sc_kernels():
    mesh = plsc.VectorSubcoreMesh(core_axis_name="c", subcore_axis_name="s")

    @functools.partial(
        pl.kernel,
        out_type=jax.ShapeDtypeStruct((2 * E, W), jnp.float32),
        mesh=mesh,
        scratch_types=[
            pltpu.VMEM((CH, 128), jnp.int32),
            pltpu.VMEM((512, W), jnp.float32),
            pltpu.SemaphoreType.DMA,
        ],
        compiler_params=pltpu.CompilerParams(use_tc_tiling_on_sc=True),
    )
    def sc_gather(table_hbm, idx_hbm, out_hbm, idx_v, rows_v, sem):
        wid = lax.axis_index("c") * 16 + lax.axis_index("s")
        pltpu.sync_copy(idx_hbm.at[wid], idx_v)
        for half in range(2):
            cps = []
            for j in range(CH // 2):
                cps.append(pltpu.async_copy(
                    table_hbm.at[idx_v.at[half * (CH // 2) + j]],
                    rows_v.at[pl.ds(j * 128, 128)], sem))
            for c in cps:
                c.wait()
            pltpu.sync_copy(
                rows_v, out_hbm.at[pl.ds(wid * BPW + half * 512, 512)])

    @functools.partial(
        pl.kernel,
        out_type=jax.ShapeDtypeStruct((2 * N, D), jnp.float32),
        mesh=mesh,
        scratch_types=[
            pltpu.VMEM((CH, 128), jnp.int32),
            pltpu.VMEM((512, D), jnp.float32),
            pltpu.VMEM_SHARED((N, D), jnp.float32),
        ],
        compiler_params=pltpu.CompilerParams(use_tc_tiling_on_sc=False),
    )
    def sc_scatter(msg_hbm, idx_hbm, zero_hbm, out_hbm, idx_v, msg_v, acc_sh):
        c = lax.axis_index("c")
        s = lax.axis_index("s")
        wid = c * 16 + s
        for q in range(4):
            pltpu.sync_copy(zero_hbm,
                            acc_sh.at[pl.ds(s * 512 + q * 128, 128)])
        plsc.subcore_barrier()
        pltpu.sync_copy(idx_hbm.at[wid], idx_v)
        for half in range(2):
            pltpu.sync_copy(
                msg_hbm.at[pl.ds(wid * BPW + half * 512, 512), pl.ds(0, D)],
                msg_v)
            for j in range(CH // 2):
                pltpu.sync_copy(msg_v.at[pl.ds(j * 128, 128)],
                                acc_sh.at[idx_v.at[half * (CH // 2) + j]],
                                add=True)
        plsc.subcore_barrier()
        pltpu.sync_copy(acc_sh.at[pl.ds(s * 512, 512)],
                        out_hbm.at[pl.ds(c * N + s * 512, 512)])

    return sc_gather, sc_scatter


def _dot(a, b):
    return lax.dot_general(a, b, (((1,), (0,)), ((), ())),
                           preferred_element_type=jnp.float32)


def _dot_t(a, b):
    return lax.dot_general(a, b, (((1,), (1,)), ((), ())),
                           preferred_element_type=jnp.float32)


def _proj_body(x_ref, w_ref, b_ref, o_ref):
    o_ref[...] = jax.nn.relu(_dot(x_ref[...], w_ref[0]) + b_ref[0, 0])


def _proj(xs, w_s, b_s):
    nb = 2 * N // RBLK
    return pl.pallas_call(
        _proj_body,
        grid=(2, nb // 2),
        in_specs=[
            pl.BlockSpec((RBLK, D), lambda b, i: (b * (nb // 2) + i, 0)),
            pl.BlockSpec((1, D, W), lambda b, i: (b, 0, 0)),
            pl.BlockSpec((1, 1, W), lambda b, i: (b, 0, 0)),
        ],
        out_specs=pl.BlockSpec((RBLK, W), lambda b, i: (b * (nb // 2) + i, 0)),
        out_shape=jax.ShapeDtypeStruct((2 * N, W), jnp.float32),
    )(xs, w_s, b_s)


def _msg_body(hs_ref, ea_ref, w1_ref, b1_ref, t2_ref, bm_ref, r_ref, s_ref,
              o_ref):
    hs = hs_ref[:, 0:D]
    zed = jax.nn.relu(_dot(ea_ref[...], w1_ref[0]) + b1_ref[0, 0])
    p = _dot(hs, t2_ref[0])
    z = _dot(zed, r_ref[...])
    o_ref[:, 0:D] = _dot(z * p, s_ref[...]) + _dot(hs, bm_ref[0])
    o_ref[:, D:W] = jnp.zeros((RBLK, W - D), jnp.float32)


def _msg(hs, ea_s, w1_s, b1_s, t2_s, bm_s, rmat, smat):
    eb = 2 * E // RBLK
    return pl.pallas_call(
        _msg_body,
        grid=(2, eb // 2),
        in_specs=[
            pl.BlockSpec((RBLK, W), lambda b, i: (b * (eb // 2) + i, 0)),
            pl.BlockSpec((RBLK, ED), lambda b, i: (b * (eb // 2) + i, 0)),
            pl.BlockSpec((1, ED, EH), lambda b, i: (b, 0, 0)),
            pl.BlockSpec((1, 1, EH), lambda b, i: (b, 0, 0)),
            pl.BlockSpec((1, D, EH * D), lambda b, i: (b, 0, 0)),
            pl.BlockSpec((1, D, D), lambda b, i: (b, 0, 0)),
            pl.BlockSpec((EH, EH * D), lambda b, i: (0, 0)),
            pl.BlockSpec((EH * D, D), lambda b, i: (0, 0)),
        ],
        out_specs=pl.BlockSpec((RBLK, W), lambda b, i: (b * (eb // 2) + i, 0)),
        out_shape=jax.ShapeDtypeStruct((2 * E, W), jnp.float32),
    )(hs, ea_s, w1_s, b1_s, t2_s, bm_s, rmat, smat)


def _gru_math(x, h, wi, wh, bi, bh):
    r = jax.nn.sigmoid(_dot_t(x, wi[0:D]) + bi[:, 0:D]
                       + _dot_t(h, wh[0:D]) + bh[:, 0:D])
    z = jax.nn.sigmoid(_dot_t(x, wi[D:2 * D]) + bi[:, D:2 * D]
                       + _dot_t(h, wh[D:2 * D]) + bh[:, D:2 * D])
    n = jnp.tanh(_dot_t(x, wi[2 * D:]) + bi[:, 2 * D:]
                 + r * (_dot_t(h, wh[2 * D:]) + bh[:, 2 * D:]))
    return (1.0 - z) * n + z * h


def _gru_body(a_ref, h_ref, bc_ref, wi_ref, wh_ref, bi_ref, bh_ref, o_ref):
    m = jax.nn.relu(a_ref[...] + bc_ref[0, 0])
    h = h_ref[:, 0:D]
    out = _gru_math(m, h, wi_ref[0], wh_ref[0], bi_ref[0], bh_ref[0])
    o_ref[:, 0:D] = out
    o_ref[:, D:W] = jnp.zeros((RBLK, W - D), jnp.float32)


def _gru_step(agg, hid, bc_s, wi_s, wh_s, bi_s, bh_s):
    nb = 2 * N // RBLK
    return pl.pallas_call(
        _gru_body,
        grid=(2, nb // 2),
        in_specs=[
            pl.BlockSpec((RBLK, D), lambda b, i: (b * (nb // 2) + i, 0)),
            pl.BlockSpec((RBLK, W), lambda b, i: (b * (nb // 2) + i, 0)),
            pl.BlockSpec((1, 1, D), lambda b, i: (b, 0, 0)),
            pl.BlockSpec((1, 3 * D, D), lambda b, i: (b, 0, 0)),
            pl.BlockSpec((1, 3 * D, D), lambda b, i: (b, 0, 0)),
            pl.BlockSpec((1, 1, 3 * D), lambda b, i: (b, 0, 0)),
            pl.BlockSpec((1, 1, 3 * D), lambda b, i: (b, 0, 0)),
        ],
        out_specs=pl.BlockSpec((RBLK, W), lambda b, i: (b * (nb // 2) + i, 0)),
        out_shape=jax.ShapeDtypeStruct((2 * N, W), jnp.float32),
    )(agg, hid, bc_s, wi_s, wh_s, bi_s, bh_s)


def _readout_body(nf_ref, ng_ref, *refs):
    wrefs = refs[:8 * TS]
    g_ref = refs[8 * TS]
    oht_sc, ms_sc, ss_sc, gf_sc, gr_sc = refs[8 * TS + 1:]

    gf_sc[...] = jnp.zeros((G, D), jnp.float32)
    def init_blk(i, _):
        row = ng_ref[pl.ds(i, 1), :]
        io = lax.broadcasted_iota(jnp.int32, (G, RBLK), 0).astype(jnp.float32)
        oht = jnp.where(io == row, 1.0, 0.0)
        oht_sc[pl.ds(i, 1)] = oht.reshape(1, G, RBLK)
        gf_sc[...] += _dot(oht, nf_ref[pl.ds(i * RBLK, RBLK), 0:D])
        return 0
    lax.fori_loop(0, NRB, init_blk, 0)

    for ts in range(TS):
        (wl_ref, bl_ref, wn_ref, bn_ref,
         wi_ref, wh_ref, bi_ref, bh_ref) = wrefs[8 * ts:8 * ts + 8]
        gf = gf_sc[...]
        dcol = lambda c, m_: lax.dot_general(
            c, m_, (((0,), (1,)), ((), ())),
            preferred_element_type=jnp.float32)
        gv = dcol(wl_ref[0:D, :], jax.nn.relu(gf))
        blv = bl_ref[0, 0]

        ms_sc[...] = jnp.full((G, 1), -1e30, jnp.float32)
        ss_sc[...] = jnp.zeros((G, 1), jnp.float32)
        gr_sc[...] = jnp.zeros((G, D), jnp.float32)
        def p1(i, _):
            oht = oht_sc[pl.ds(i, 1)].reshape(G, RBLK)
            nf = nf_ref[pl.ds(i * RBLK, RBLK), 0:D]
            lg = _dot(gv, oht) + dcol(wl_ref[D:2 * D, :], nf) + blv
            lg = jnp.where(lg >= 0, lg, 0.01 * lg)
            masked = jnp.where(oht > 0.5, lg, -1e30)
            m_old = ms_sc[...]
            m_new = jnp.maximum(m_old,
                                jnp.max(masked, axis=1, keepdims=True))
            scale = jnp.exp(m_old - m_new)
            mb = jnp.sum(oht * m_new, axis=0, keepdims=True)
            ex = jnp.exp(lg - mb)
            exm = oht * ex
            hv = _dot(nf, wn_ref[...]) + bn_ref[...]
            ms_sc[...] = m_new
            ss_sc[...] = ss_sc[...] * scale + jnp.sum(exm, axis=1,
                                                      keepdims=True)
            gr_sc[...] = gr_sc[...] * scale + _dot(exm, hv)
            return 0
        lax.fori_loop(0, NRB, p1, 0)
        ssv = ss_sc[...]
        grep = jnp.where(ssv > 0, gr_sc[...] / ssv, 0.0)
        grep = jnp.where(grep > 0, grep, jnp.exp(jnp.minimum(grep, 0.0)) - 1.0)

        gf_sc[...] = _gru_math(grep, gf, wi_ref, wh_ref,
                               bi_ref[...], bh_ref[...])

    g_ref[...] = gf_sc[...]


def _readout(node, branch, ng8, wts):
    nspec = [
        pl.BlockSpec((N, W), lambda g: (branch, 0)),
        pl.BlockSpec((NRB, RBLK), lambda g: (0, 0)),
    ] + [pl.BlockSpec(w.shape, lambda g, nd=w.ndim: (0,) * nd) for w in wts]
    return pl.pallas_call(
        _readout_body,
        grid=(1,),
        in_specs=nspec,
        out_specs=pl.BlockSpec((G, D), lambda g: (0, 0)),
        out_shape=jax.ShapeDtypeStruct((G, D), jnp.float32),
        scratch_shapes=[
            pltpu.VMEM((NRB, G, RBLK), jnp.float32),
            pltpu.VMEM((G, 1), jnp.float32),
            pltpu.VMEM((G, 1), jnp.float32),
            pltpu.VMEM((G, D), jnp.float32),
            pltpu.VMEM((G, D), jnp.float32),
        ],
    )(node, ng8, *wts)


def _pred_body(g1_ref, g2_ref, mc_ref, w1a_ref, w1b_ref, w1c_ref, b1_ref,
               gm_ref, bt_ref, w2_ref, b2_ref, o_ref):
    h = jax.nn.relu(_dot(g1_ref[...], w1a_ref[...])
                    + _dot(g2_ref[...], w1b_ref[...])
                    + _dot(mc_ref[...], w1c_ref[...]) + b1_ref[...])
    h = h * (gm_ref[...] / jnp.sqrt(1.0 + 1e-5)) + bt_ref[...]
    o_ref[...] = _dot(h, w2_ref[...]) + b2_ref[...]


def _pred(g1, g2, mc, w1a, w1b, w1c, b1, gm, bt, w2, b2):
    return pl.pallas_call(
        _pred_body,
        out_shape=jax.ShapeDtypeStruct((G, 2), jnp.float32),
    )(g1, g2, mc, w1a, w1b, w1c, b1, gm, bt, w2, b2)


def kernel(x1, edge_index1, edge_attr1, node2graph1,
           x2, edge_index2, edge_attr2, node2graph2, mc, params):
    m1, m2 = params['mpnn1'], params['mpnn2']
    f32 = jnp.float32

    xs = jnp.concatenate([x1, x2], axis=0)
    ea_s = jnp.concatenate([edge_attr1, edge_attr2], axis=0)
    src3 = jnp.concatenate(
        [edge_index1[0], edge_index2[0] + N]).reshape(NW, CH, 128)
    dst3 = jnp.concatenate(
        [edge_index1[1], edge_index2[1]]).reshape(NW, CH, 128)
    zeros128 = jnp.zeros((128, D), f32)

    def padw(a):
        return jnp.pad(a, ((0, 0), (0, W - D)))

    wp_s = jnp.stack([padw(m1['Wp']), padw(m2['Wp'])])
    bp_s = jnp.stack([jnp.pad(m1['bp'], (0, W - D)),
                      jnp.pad(m2['bp'], (0, W - D))]).reshape(2, 1, W)
    w1_s = jnp.stack([m1['W1'], m2['W1']])
    b1_s = jnp.stack([m1['b1'], m2['b1']]).reshape(2, 1, EH)

    def t2(m):
        return m['W2'].reshape(EH, D, D).transpose(1, 0, 2).reshape(D, EH * D)

    t2_s = jnp.stack([t2(m1), t2(m2)])
    bm_s = jnp.stack([m1['b2'].reshape(D, D), m2['b2'].reshape(D, D)])
    rmat = jnp.repeat(jnp.eye(EH, dtype=f32), D, axis=1)
    smat = jnp.tile(jnp.eye(D, dtype=f32), (EH, 1))
    bc_s = jnp.stack([m1['b_conv'], m2['b_conv']]).reshape(2, 1, D)
    wi_s = jnp.stack([m1['gru']['W_ih'], m2['gru']['W_ih']])
    wh_s = jnp.stack([m1['gru']['W_hh'], m2['gru']['W_hh']])
    bi_s = jnp.stack([m1['gru']['b_ih'], m2['gru']['b_ih']]).reshape(2, 1, 3 * D)
    bh_s = jnp.stack([m1['gru']['b_hh'], m2['gru']['b_hh']]).reshape(2, 1, 3 * D)

    def ro_wts(plist):
        wts = []
        for p in plist:
            wts += [p['Wl'], p['bl'].reshape(1, 1), p['Wn'],
                    p['bn'].reshape(1, D),
                    p['gru']['W_ih'], p['gru']['W_hh'],
                    p['gru']['b_ih'].reshape(1, 3 * D),
                    p['gru']['b_hh'].reshape(1, 3 * D)]
        return wts

    sc_gather, sc_scatter = _sc_kernels()
    node = _proj(xs, wp_s, bp_s)
    hidden = node
    for _ in range(STEPS):
        hs = sc_gather(node, src3)
        msg = _msg(hs, ea_s, w1_s, b1_s, t2_s, bm_s, rmat, smat)
        agg = sc_scatter(msg, dst3, zeros128)
        node = _gru_step(agg, hidden, bc_s, wi_s, wh_s, bi_s, bh_s)
        hidden = node

    ng81 = node2graph1.astype(f32).reshape(NRB, RBLK)
    ng82 = node2graph2.astype(f32).reshape(NRB, RBLK)
    r1 = _readout(node, 0, ng81, ro_wts(params['ro1']))
    r2 = _readout(node, 1, ng82, ro_wts(params['ro2']))

    pp = params['pred']
    return _pred(r1, r2, mc,
                 pp['W1'][0:D], pp['W1'][D:2 * D], pp['W1'][2 * D:],
                 pp['b1'].reshape(1, HID), pp['gamma'].reshape(1, HID),
                 pp['beta'].reshape(1, HID), pp['W2'], pp['b2'].reshape(1, 2))


# speedup vs baseline: 7.1897x; 1.0419x over previous
import functools

import jax
import jax.numpy as jnp
from jax import lax
from jax.experimental import pallas as pl
from jax.experimental.pallas import tpu as pltpu
from jax.experimental.pallas import tpu_sc as plsc

N = 8192
E = 16384
G = 256
D = 64
ED = 12
EH = 12
HID = 256
STEPS = 3
TS = 6

W = 128
NW = 32
BPW = 2 * E // NW
CH = BPW // 128
RBLK = 1024
NBLK = N // RBLK
NRB = NBLK


@functools.cache
def _sc_kernels():
    mesh = plsc.VectorSubcoreMesh(core_axis_name="c", subcore_axis_name="s")

    @functools.partial(
        pl.kernel,
        out_type=jax.ShapeDtypeStruct((2 * E, W), jnp.float32),
        mesh=mesh,
        scratch_types=[
            pltpu.VMEM((CH, 128), jnp.int32),
            pltpu.VMEM((512, W), jnp.float32),
            pltpu.SemaphoreType.DMA,
        ],
        compiler_params=pltpu.CompilerParams(use_tc_tiling_on_sc=True),
    )
    def sc_gather(table_hbm, idx_hbm, out_hbm, idx_v, rows_v, sem):
        wid = lax.axis_index("c") * 16 + lax.axis_index("s")
        pltpu.sync_copy(idx_hbm.at[wid], idx_v)
        for half in range(2):
            cps = []
            for j in range(CH // 2):
                cps.append(pltpu.async_copy(
                    table_hbm.at[idx_v.at[half * (CH // 2) + j]],
                    rows_v.at[pl.ds(j * 128, 128)], sem))
            for c in cps:
                c.wait()
            pltpu.sync_copy(
                rows_v, out_hbm.at[pl.ds(wid * BPW + half * 512, 512)])

    @functools.partial(
        pl.kernel,
        out_type=jax.ShapeDtypeStruct((2 * N, D), jnp.float32),
        mesh=mesh,
        scratch_types=[
            pltpu.VMEM((CH, 128), jnp.int32),
            pltpu.VMEM((512, D), jnp.float32),
            pltpu.VMEM_SHARED((N, D), jnp.float32),
        ],
        compiler_params=pltpu.CompilerParams(use_tc_tiling_on_sc=False),
    )
    def sc_scatter(msg_hbm, idx_hbm, zero_hbm, out_hbm, idx_v, msg_v, acc_sh):
        c = lax.axis_index("c")
        s = lax.axis_index("s")
        wid = c * 16 + s
        for q in range(4):
            pltpu.sync_copy(zero_hbm,
                            acc_sh.at[pl.ds(s * 512 + q * 128, 128)])
        plsc.subcore_barrier()
        pltpu.sync_copy(idx_hbm.at[wid], idx_v)
        for half in range(2):
            pltpu.sync_copy(
                msg_hbm.at[pl.ds(wid * BPW + half * 512, 512), pl.ds(0, D)],
                msg_v)
            for j in range(CH // 2):
                pltpu.sync_copy(msg_v.at[pl.ds(j * 128, 128)],
                                acc_sh.at[idx_v.at[half * (CH // 2) + j]],
                                add=True)
        plsc.subcore_barrier()
        pltpu.sync_copy(acc_sh.at[pl.ds(s * 512, 512)],
                        out_hbm.at[pl.ds(c * N + s * 512, 512)])

    return sc_gather, sc_scatter


def _dot(a, b):
    return lax.dot_general(a, b, (((1,), (0,)), ((), ())),
                           preferred_element_type=jnp.float32)


def _dot_t(a, b):
    return lax.dot_general(a, b, (((1,), (1,)), ((), ())),
                           preferred_element_type=jnp.float32)


def _dot_lt(a, b):
    return lax.dot_general(a, b, (((0,), (0,)), ((), ())),
                           preferred_element_type=jnp.float32)


def _proj_body(x_ref, w_ref, b_ref, o_ref):
    o_ref[...] = jax.nn.relu(_dot_lt(x_ref[...], w_ref[0]) + b_ref[0, 0])


def _proj(xst, w_s, b_s):
    nb = 2 * N // RBLK
    return pl.pallas_call(
        _proj_body,
        grid=(2, nb // 2),
        in_specs=[
            pl.BlockSpec((D, RBLK), lambda b, i: (0, b * (nb // 2) + i)),
            pl.BlockSpec((1, D, W), lambda b, i: (b, 0, 0)),
            pl.BlockSpec((1, 1, W), lambda b, i: (b, 0, 0)),
        ],
        out_specs=pl.BlockSpec((RBLK, W), lambda b, i: (b * (nb // 2) + i, 0)),
        out_shape=jax.ShapeDtypeStruct((2 * N, W), jnp.float32),
    )(xst, w_s, b_s)


def _msg_body(hs_ref, ea_ref, w1_ref, b1_ref, t2_ref, bm_ref, r_ref, s_ref,
              o_ref):
    hs = hs_ref[:, 0:D]
    zed = jax.nn.relu(_dot_lt(ea_ref[...], w1_ref[0]) + b1_ref[0, 0])
    p = _dot(hs, t2_ref[0])
    z = _dot(zed, r_ref[...])
    o_ref[:, 0:D] = _dot(z * p, s_ref[...]) + _dot(hs, bm_ref[0])
    o_ref[:, D:W] = jnp.zeros((RBLK, W - D), jnp.float32)


def _msg(hs, ea_s, w1_s, b1_s, t2_s, bm_s, rmat, smat):
    eb = 2 * E // RBLK
    return pl.pallas_call(
        _msg_body,
        grid=(2, eb // 2),
        in_specs=[
            pl.BlockSpec((RBLK, W), lambda b, i: (b * (eb // 2) + i, 0)),
            pl.BlockSpec((ED, RBLK), lambda b, i: (0, b * (eb // 2) + i)),
            pl.BlockSpec((1, ED, EH), lambda b, i: (b, 0, 0)),
            pl.BlockSpec((1, 1, EH), lambda b, i: (b, 0, 0)),
            pl.BlockSpec((1, D, EH * D), lambda b, i: (b, 0, 0)),
            pl.BlockSpec((1, D, D), lambda b, i: (b, 0, 0)),
            pl.BlockSpec((EH, EH * D), lambda b, i: (0, 0)),
            pl.BlockSpec((EH * D, D), lambda b, i: (0, 0)),
        ],
        out_specs=pl.BlockSpec((RBLK, W), lambda b, i: (b * (eb // 2) + i, 0)),
        out_shape=jax.ShapeDtypeStruct((2 * E, W), jnp.float32),
    )(hs, ea_s, w1_s, b1_s, t2_s, bm_s, rmat, smat)


def _gru_math(x, h, wi, wh, bi, bh):
    r = jax.nn.sigmoid(_dot_t(x, wi[0:D]) + bi[:, 0:D]
                       + _dot_t(h, wh[0:D]) + bh[:, 0:D])
    z = jax.nn.sigmoid(_dot_t(x, wi[D:2 * D]) + bi[:, D:2 * D]
                       + _dot_t(h, wh[D:2 * D]) + bh[:, D:2 * D])
    n = jnp.tanh(_dot_t(x, wi[2 * D:]) + bi[:, 2 * D:]
                 + r * (_dot_t(h, wh[2 * D:]) + bh[:, 2 * D:]))
    return (1.0 - z) * n + z * h


def _gru_body(a_ref, h_ref, bc_ref, wi_ref, wh_ref, bi_ref, bh_ref, o_ref):
    m = jax.nn.relu(a_ref[...] + bc_ref[0, 0])
    h = h_ref[:, 0:D]
    out = _gru_math(m, h, wi_ref[0], wh_ref[0], bi_ref[0], bh_ref[0])
    o_ref[:, 0:D] = out
    o_ref[:, D:W] = jnp.zeros((RBLK, W - D), jnp.float32)


def _gru_step(agg, hid, bc_s, wi_s, wh_s, bi_s, bh_s):
    nb = 2 * N // RBLK
    return pl.pallas_call(
        _gru_body,
        grid=(2, nb // 2),
        in_specs=[
            pl.BlockSpec((RBLK, D), lambda b, i: (b * (nb // 2) + i, 0)),
            pl.BlockSpec((RBLK, W), lambda b, i: (b * (nb // 2) + i, 0)),
            pl.BlockSpec((1, 1, D), lambda b, i: (b, 0, 0)),
            pl.BlockSpec((1, 3 * D, D), lambda b, i: (b, 0, 0)),
            pl.BlockSpec((1, 3 * D, D), lambda b, i: (b, 0, 0)),
            pl.BlockSpec((1, 1, 3 * D), lambda b, i: (b, 0, 0)),
            pl.BlockSpec((1, 1, 3 * D), lambda b, i: (b, 0, 0)),
        ],
        out_specs=pl.BlockSpec((RBLK, W), lambda b, i: (b * (nb // 2) + i, 0)),
        out_shape=jax.ShapeDtypeStruct((2 * N, W), jnp.float32),
    )(agg, hid, bc_s, wi_s, wh_s, bi_s, bh_s)


def _readout_body(nf_ref, ng_ref, *refs):
    wrefs = refs[:8 * TS]
    g_ref = refs[8 * TS]
    ms_sc, ss_sc, gf_sc, gr_sc = refs[8 * TS + 1:]

    def oht_at(i):
        row = ng_ref[pl.ds(i, 1), :]
        io = lax.broadcasted_iota(jnp.int32, (G, RBLK), 0).astype(jnp.float32)
        return jnp.where(io == row, 1.0, 0.0)

    gf_sc[...] = jnp.zeros((G, D), jnp.float32)
    def init_blk(i, _):
        gf_sc[...] += _dot(oht_at(i), nf_ref[pl.ds(i * RBLK, RBLK), 0:D])
        return 0
    lax.fori_loop(0, NRB, init_blk, 0)

    for ts in range(TS):
        (wl_ref, bl_ref, wn_ref, bn_ref,
         wi_ref, wh_ref, bi_ref, bh_ref) = wrefs[8 * ts:8 * ts + 8]
        gf = gf_sc[...]
        gv = _dot_t(wl_ref[:, 0:D], jax.nn.relu(gf))
        blv = bl_ref[0, 0]

        ms_sc[...] = jnp.full((G, 1), -1e30, jnp.float32)
        ss_sc[...] = jnp.zeros((G, 1), jnp.float32)
        gr_sc[...] = jnp.zeros((G, D), jnp.float32)
        def p1(i, _):
            oht = oht_at(i)
            nf = nf_ref[pl.ds(i * RBLK, RBLK), 0:D]
            lg = _dot(gv, oht) + _dot_t(wl_ref[:, D:2 * D], nf) + blv
            lg = jnp.where(lg >= 0, lg, 0.01 * lg)
            masked = jnp.where(oht > 0.5, lg, -1e30)
            m_old = ms_sc[...]
            m_new = jnp.maximum(m_old,
                                jnp.max(masked, axis=1, keepdims=True))
            scale = jnp.exp(m_old - m_new)
            mb = jnp.sum(oht * m_new, axis=0, keepdims=True)
            ex = jnp.exp(lg - mb)
            exm = oht * ex
            hv = _dot(nf, wn_ref[...]) + bn_ref[...]
            ms_sc[...] = m_new
            ss_sc[...] = ss_sc[...] * scale + jnp.sum(exm, axis=1,
                                                      keepdims=True)
            gr_sc[...] = gr_sc[...] * scale + _dot(exm, hv)
            return 0
        lax.fori_loop(0, NRB, p1, 0)
        ssv = ss_sc[...]
        grep = jnp.where(ssv > 0, gr_sc[...] / ssv, 0.0)
        grep = jnp.where(grep > 0, grep, jnp.exp(jnp.minimum(grep, 0.0)) - 1.0)

        gf_sc[...] = _gru_math(grep, gf, wi_ref, wh_ref,
                               bi_ref[...], bh_ref[...])

    g_ref[...] = gf_sc[...]


def _readout(node, branch, ng8, wts):
    nspec = [
        pl.BlockSpec((N, W), lambda g: (branch, 0)),
        pl.BlockSpec((NRB, RBLK), lambda g: (0, 0)),
    ] + [pl.BlockSpec(w.shape, lambda g, nd=w.ndim: (0,) * nd) for w in wts]
    return pl.pallas_call(
        _readout_body,
        grid=(1,),
        in_specs=nspec,
        out_specs=pl.BlockSpec((G, D), lambda g: (0, 0)),
        out_shape=jax.ShapeDtypeStruct((G, D), jnp.float32),
        scratch_shapes=[
            pltpu.VMEM((G, 1), jnp.float32),
            pltpu.VMEM((G, 1), jnp.float32),
            pltpu.VMEM((G, D), jnp.float32),
            pltpu.VMEM((G, D), jnp.float32),
        ],
    )(node, ng8, *wts)


def _pred_body(g1_ref, g2_ref, mc_ref, w1a_ref, w1b_ref, w1c_ref, b1_ref,
               gm_ref, bt_ref, w2_ref, b2_ref, o_ref):
    h = jax.nn.relu(_dot(g1_ref[...], w1a_ref[...])
                    + _dot(g2_ref[...], w1b_ref[...])
                    + _dot(mc_ref[...], w1c_ref[...]) + b1_ref[...])
    h = h * (gm_ref[...] / jnp.sqrt(1.0 + 1e-5)) + bt_ref[...]
    o_ref[...] = _dot(h, w2_ref[...]) + b2_ref[...]


def _pred(g1, g2, mc, w1a, w1b, w1c, b1, gm, bt, w2, b2):
    return pl.pallas_call(
        _pred_body,
        out_shape=jax.ShapeDtypeStruct((G, 2), jnp.float32),
    )(g1, g2, mc, w1a, w1b, w1c, b1, gm, bt, w2, b2)


def kernel(x1, edge_index1, edge_attr1, node2graph1,
           x2, edge_index2, edge_attr2, node2graph2, mc, params):
    m1, m2 = params['mpnn1'], params['mpnn2']
    f32 = jnp.float32

    xst = jnp.concatenate([x1.T, x2.T], axis=1)
    eat = jnp.concatenate([edge_attr1.T, edge_attr2.T], axis=1)
    src3 = jnp.concatenate(
        [edge_index1[0], edge_index2[0] + N]).reshape(NW, CH, 128)
    dst3 = jnp.concatenate(
        [edge_index1[1], edge_index2[1]]).reshape(NW, CH, 128)
    zeros128 = jnp.zeros((128, D), f32)

    def padw(a):
        return jnp.pad(a, ((0, 0), (0, W - D)))

    wp_s = jnp.stack([padw(m1['Wp']), padw(m2['Wp'])])
    bp_s = jnp.stack([jnp.pad(m1['bp'], (0, W - D)),
                      jnp.pad(m2['bp'], (0, W - D))]).reshape(2, 1, W)
    w1_s = jnp.stack([m1['W1'], m2['W1']])
    b1_s = jnp.stack([m1['b1'], m2['b1']]).reshape(2, 1, EH)

    def t2(m):
        return m['W2'].reshape(EH, D, D).transpose(1, 0, 2).reshape(D, EH * D)

    t2_s = jnp.stack([t2(m1), t2(m2)])
    bm_s = jnp.stack([m1['b2'].reshape(D, D), m2['b2'].reshape(D, D)])
    rmat = jnp.repeat(jnp.eye(EH, dtype=f32), D, axis=1)
    smat = jnp.tile(jnp.eye(D, dtype=f32), (EH, 1))
    bc_s = jnp.stack([m1['b_conv'], m2['b_conv']]).reshape(2, 1, D)
    wi_s = jnp.stack([m1['gru']['W_ih'], m2['gru']['W_ih']])
    wh_s = jnp.stack([m1['gru']['W_hh'], m2['gru']['W_hh']])
    bi_s = jnp.stack([m1['gru']['b_ih'], m2['gru']['b_ih']]).reshape(2, 1, 3 * D)
    bh_s = jnp.stack([m1['gru']['b_hh'], m2['gru']['b_hh']]).reshape(2, 1, 3 * D)

    def ro_wts(plist):
        wts = []
        for p in plist:
            wts += [p['Wl'].T, p['bl'].reshape(1, 1), p['Wn'],
                    p['bn'].reshape(1, D),
                    p['gru']['W_ih'], p['gru']['W_hh'],
                    p['gru']['b_ih'].reshape(1, 3 * D),
                    p['gru']['b_hh'].reshape(1, 3 * D)]
        return wts

    sc_gather, sc_scatter = _sc_kernels()
    node = _proj(xst, wp_s, bp_s)
    hidden = node
    for _ in range(STEPS):
        hs = sc_gather(node, src3)
        msg = _msg(hs, eat, w1_s, b1_s, t2_s, bm_s, rmat, smat)
        agg = sc_scatter(msg, dst3, zeros128)
        node = _gru_step(agg, hidden, bc_s, wi_s, wh_s, bi_s, bh_s)
        hidden = node

    ng81 = node2graph1.astype(f32).reshape(NRB, RBLK)
    ng82 = node2graph2.astype(f32).reshape(NRB, RBLK)
    r1 = _readout(node, 0, ng81, ro_wts(params['ro1']))
    r2 = _readout(node, 1, ng82, ro_wts(params['ro2']))

    pp = params['pred']
    return _pred(r1, r2, mc,
                 pp['W1'][0:D], pp['W1'][D:2 * D], pp['W1'][2 * D:],
                 pp['b1'].reshape(1, HID), pp['gamma'].reshape(1, HID),
                 pp['beta'].reshape(1, HID), pp['W2'], pp['b2'].reshape(1, 2))
